# Initial kernel scaffold; baseline (speedup 1.0000x reference)
import jax
import jax.numpy as jnp
from jax import lax
from jax.experimental import pallas as pl
from jax.experimental.pallas import tpu as pltpu
from jax.experimental.pallas import tpu_sc as plsc

_N = 10000
_R = 3
_D = 128
_K = 128
_NSUB = 16
_NCORE = 2
_NHALF = _N // _NCORE
_SREL = _NHALF + 8
_ACC_RAW = _R * _SREL
_ACC_ROWS = ((_ACC_RAW + 64 + 15) // 16) * 16
_TRASH0 = _ACC_RAW
_CNT_ROWS = _ACC_ROWS // 16
_ROWS_PER_TILE = _ACC_ROWS // _NSUB
_CROWS_PER_TILE = _CNT_ROWS // _NSUB


def _make_sc_agg(with_cnt, n_chunks_per_tile):
    out_types = [jax.ShapeDtypeStruct((_NCORE, _ACC_ROWS, _D), jnp.float32)]
    if with_cnt:
        out_types.append(
            jax.ShapeDtypeStruct((_NCORE, _CNT_ROWS, 16), jnp.float32))

    scratch = [
        pltpu.VMEM((3, _K), jnp.int32),
        pltpu.VMEM((1, _K), jnp.int32),
        pltpu.VMEM((1, _K), jnp.int32),
        pltpu.VMEM((_K, _D), jnp.float32),
        pltpu.VMEM((_K, 16), jnp.float32),
        pltpu.VMEM((128, _D), jnp.float32),
        pltpu.VMEM((_CROWS_PER_TILE, 16), jnp.float32),
        pltpu.VMEM_SHARED((_ACC_ROWS, _D), jnp.float32),
        pltpu.VMEM_SHARED((_CNT_ROWS, 16), jnp.float32),
    ]

    def body(src_hbm, dst_hbm, typ_hbm, table_hbm, *rest):
        if with_cnt:
            sum_out, cnt_out = rest[0], rest[1]
            (ebuf, segbuf, crowbuf, rows, onehot, zbuf, zcnt,
             acc, cntacc) = rest[2:]
        else:
            sum_out = rest[0]
            cnt_out = None
            (ebuf, segbuf, crowbuf, rows, onehot, zbuf, zcnt,
             acc, cntacc) = rest[1:]
        c = lax.axis_index("c")
        s = lax.axis_index("s")
        lo = c * _NHALF

        def zrow(i, _):
            for l in range(_D // 16):
                zbuf[i, pl.ds(l * 16, 16)] = jnp.zeros((16,), jnp.float32)
            return 0
        lax.fori_loop(0, 128, zrow, 0)

        def zcrow(i, _):
            zcnt[i, pl.ds(0, 16)] = jnp.zeros((16,), jnp.float32)
            return 0
        lax.fori_loop(0, _CROWS_PER_TILE, zcrow, 0)

        for k in range(_ROWS_PER_TILE // 128):
            pltpu.sync_copy(zbuf,
                            acc.at[pl.ds(s * _ROWS_PER_TILE + k * 128, 128)])
        rem = _ROWS_PER_TILE % 128
        if rem:
            pltpu.sync_copy(
                zbuf.at[pl.ds(0, rem)],
                acc.at[pl.ds(s * _ROWS_PER_TILE
                             + (_ROWS_PER_TILE // 128) * 128, rem)])
        pltpu.sync_copy(zcnt, cntacc.at[pl.ds(s * _CROWS_PER_TILE,
                                              _CROWS_PER_TILE)])
        plsc.subcore_barrier()

        lane = lax.iota(jnp.int32, 16)

        def chunk_body(i, _):
            ch = s + i * _NSUB
            base = ch * _K
            pltpu.sync_copy(src_hbm.at[pl.ds(base, _K)], ebuf.at[0])
            pltpu.sync_copy(dst_hbm.at[pl.ds(base, _K)], ebuf.at[1])
            pltpu.sync_copy(typ_hbm.at[pl.ds(base, _K)], ebuf.at[2])
            for g in range(_K // 16):
                d = ebuf[1, pl.ds(g * 16, 16)]
                t = ebuf[2, pl.ds(g * 16, 16)]
                dl = d - lo
                inrange = (dl >= 0) & (dl < _NHALF)
                trash = _TRASH0 + ((lane + g * 16) & 63)
                seg = jnp.where(inrange, t * _SREL + dl, trash)
                segbuf[0, pl.ds(g * 16, 16)] = seg
                if with_cnt:
                    crowbuf[0, pl.ds(g * 16, 16)] = seg >> 4
            pltpu.sync_copy(table_hbm.at[ebuf.at[0]], rows)
            pltpu.sync_copy(rows, acc.at[segbuf.at[0]], add=True)
            if with_cnt:
                def zoh(r2, _2):
                    onehot[r2, pl.ds(0, 16)] = jnp.zeros((16,), jnp.float32)
                    return 0
                lax.fori_loop(0, _K, zoh, 0)
                for g in range(_K // 16):
                    segv = segbuf[0, pl.ds(g * 16, 16)]
                    col = segv & 15
                    rowi = lane + g * 16
                    plsc.addupdate_scatter(
                        onehot, [rowi, col], jnp.ones((16,), jnp.float32))
                pltpu.sync_copy(onehot, cntacc.at[crowbuf.at[0]], add=True)
            return 0

        lax.fori_loop(0, n_chunks_per_tile, chunk_body, 0)
        plsc.subcore_barrier()

        pltpu.sync_copy(
            acc.at[pl.ds(s * _ROWS_PER_TILE, _ROWS_PER_TILE)],
            sum_out.at[c, pl.ds(s * _ROWS_PER_TILE, _ROWS_PER_TILE)])
        if with_cnt:
            pltpu.sync_copy(
                cntacc.at[pl.ds(s * _CROWS_PER_TILE, _CROWS_PER_TILE)],
                cnt_out.at[c, pl.ds(s * _CROWS_PER_TILE, _CROWS_PER_TILE)])

    mesh = plsc.VectorSubcoreMesh(core_axis_name="c", subcore_axis_name="s")
    return pl.kernel(body, out_type=tuple(out_types), mesh=mesh,
                     scratch_types=scratch)


def _make_dense(relu, n, block):
    grid = (n // block,)

    def body(xb, s0, s1, s2, c0, c1, c2, w0, w1, w2, rt, bb, ob):
        acc = jnp.dot(xb[...], rt[...], preferred_element_type=jnp.float32)
        acc = acc + jnp.dot(s0[...] / jnp.maximum(c0[...], 1.0), w0[...],
                            preferred_element_type=jnp.float32)
        acc = acc + jnp.dot(s1[...] / jnp.maximum(c1[...], 1.0), w1[...],
                            preferred_element_type=jnp.float32)
        acc = acc + jnp.dot(s2[...] / jnp.maximum(c2[...], 1.0), w2[...],
                            preferred_element_type=jnp.float32)
        acc = acc + bb[...]
        if relu:
            acc = jnp.maximum(acc, 0.0)
        ob[...] = acc

    node_spec = pl.BlockSpec((block, _D), lambda i: (i, 0))
    cnt_spec = pl.BlockSpec((block, 1), lambda i: (i, 0))
    w_spec = pl.BlockSpec((_D, _D), lambda i: (0, 0))
    b_spec = pl.BlockSpec((1, _D), lambda i: (0, 0))
    return pl.pallas_call(
        body,
        grid=grid,
        in_specs=[node_spec, node_spec, node_spec, node_spec,
                  cnt_spec, cnt_spec, cnt_spec,
                  w_spec, w_spec, w_spec, w_spec, b_spec],
        out_specs=node_spec,
        out_shape=jax.ShapeDtypeStruct((n, _D), jnp.float32),
    )


def kernel(x, edge_index, edge_type, W1, root1, b1, W2, root2, b2):
    e = edge_index.shape[1]
    chunks_total = -(-e // _K)
    n_chunks_per_tile = -(-chunks_total // _NSUB)
    e_pad = n_chunks_per_tile * _NSUB * _K
    pad = e_pad - e

    src = edge_index[0]
    dst = edge_index[1]
    typ = edge_type
    if pad:
        src = jnp.concatenate([src, jnp.zeros((pad,), jnp.int32)])
        dst = jnp.concatenate([dst, jnp.full((pad,), -1, jnp.int32)])
        typ = jnp.concatenate([typ, jnp.zeros((pad,), jnp.int32)])

    sc_agg_cnt = _make_sc_agg(True, n_chunks_per_tile)
    sc_agg = _make_sc_agg(False, n_chunks_per_tile)
    dense_relu = _make_dense(True, _N, 1000)
    dense = _make_dense(False, _N, 1000)

    def split_sums(sums):
        return [sums[:, r * _SREL:r * _SREL + _NHALF, :].reshape(_N, _D)
                for r in range(_R)]

    sum1, cnt = sc_agg_cnt(src, dst, typ, x)
    cflat = cnt.reshape(_NCORE, _CNT_ROWS * 16)
    cnts = [cflat[:, r * _SREL:r * _SREL + _NHALF].reshape(_N, 1)
            for r in range(_R)]
    s1 = split_sums(sum1)
    bb1 = b1.reshape(1, _D)
    h1 = dense_relu(x, s1[0], s1[1], s1[2], cnts[0], cnts[1], cnts[2],
                    W1[0], W1[1], W1[2], root1, bb1)

    sum2 = sc_agg(src, dst, typ, h1)
    s2 = split_sums(sum2)
    bb2 = b2.reshape(1, _D)
    out = dense(h1, s2[0], s2[1], s2[2], cnts[0], cnts[1], cnts[2],
                W2[0], W2[1], W2[2], root2, bb2)
    return out


# speedup vs baseline: 5.5581x; 5.5581x over previous
import jax
import jax.numpy as jnp
from jax import lax
from jax.experimental import pallas as pl
from jax.experimental.pallas import tpu as pltpu
from jax.experimental.pallas import tpu_sc as plsc

_N = 10000
_R = 3
_D = 128
_K = 64
_NSUB = 16
_NCORE = 2
_NHALF = _N // _NCORE
_SREL = _NHALF + 8
_ACC_RAW = _R * _SREL
_ACC_ROWS = ((_ACC_RAW + 64 + 127) // 128) * 128
_TRASH0 = _ACC_RAW
_ROWS_PER_TILE = _ACC_ROWS // _NSUB


def _seg_groups(ebuf, segbuf, lo, lane):
    for g in range(_K // 16):
        d = ebuf[0, pl.ds(g * 16, 16)]
        t = ebuf[1, pl.ds(g * 16, 16)]
        dl = d - lo
        inrange = (dl >= 0) & (dl < _NHALF)
        trash = _TRASH0 + ((lane + g * 16) & 63)
        segbuf[0, pl.ds(g * 16, 16)] = jnp.where(inrange, t * _SREL + dl,
                                                 trash)


def _make_sc_counts(n_chunks_per_tile):
    scratch = [
        pltpu.VMEM((2, _K), jnp.int32),
        pltpu.VMEM((1, _K), jnp.int32),
        pltpu.VMEM((_K, _D), jnp.float32),
        pltpu.VMEM_SHARED((_ACC_ROWS, _D), jnp.float32),
    ]

    def body(dst_hbm, typ_hbm, cnt_out, ebuf, segbuf, rows, cntacc):
        c = lax.axis_index("c")
        s = lax.axis_index("s")
        lo = c * _NHALF
        lane = lax.iota(jnp.int32, 16)

        def zrow(i, _):
            for l in range(_D // 16):
                rows[i, pl.ds(l * 16, 16)] = jnp.zeros((16,), jnp.float32)
            return 0
        lax.fori_loop(0, _K, zrow, 0)
        for k in range(_ROWS_PER_TILE // _K):
            pltpu.sync_copy(rows,
                            cntacc.at[pl.ds(s * _ROWS_PER_TILE + k * _K,
                                            _K)])
        rem = _ROWS_PER_TILE % _K
        if rem:
            pltpu.sync_copy(
                rows.at[pl.ds(0, rem)],
                cntacc.at[pl.ds(s * _ROWS_PER_TILE
                                + (_ROWS_PER_TILE // _K) * _K, rem)])

        def orow(i, _):
            for l in range(_D // 16):
                rows[i, pl.ds(l * 16, 16)] = jnp.ones((16,), jnp.float32)
            return 0
        lax.fori_loop(0, _K, orow, 0)
        plsc.subcore_barrier()

        def chunk_body(i, _):
            base = (s + i * _NSUB) * _K
            pltpu.sync_copy(dst_hbm.at[pl.ds(base, _K)], ebuf.at[0])
            pltpu.sync_copy(typ_hbm.at[pl.ds(base, _K)], ebuf.at[1])
            _seg_groups(ebuf, segbuf, lo, lane)
            pltpu.sync_copy(rows, cntacc.at[segbuf.at[0]], add=True)
            return 0

        lax.fori_loop(0, n_chunks_per_tile, chunk_body, 0)
        plsc.subcore_barrier()

        pltpu.sync_copy(
            cntacc.at[pl.ds(s * _ROWS_PER_TILE, _ROWS_PER_TILE)],
            cnt_out.at[c, pl.ds(s * _ROWS_PER_TILE, _ROWS_PER_TILE)])

    mesh = plsc.VectorSubcoreMesh(core_axis_name="c", subcore_axis_name="s")
    return pl.kernel(
        body,
        out_type=jax.ShapeDtypeStruct((_NCORE, _ACC_ROWS, _D), jnp.float32),
        mesh=mesh, scratch_types=scratch)


def _make_sc_sums(n_chunks_per_tile):
    scratch = [
        pltpu.VMEM((2, _K), jnp.int32),
        pltpu.VMEM((_K,), jnp.int32),
        pltpu.VMEM((1, _K), jnp.int32),
        pltpu.VMEM((_K, _D), jnp.float32),
        pltpu.SemaphoreType.DMA,
        pltpu.VMEM_SHARED((_ACC_ROWS, _D), jnp.float32),
    ]

    def body(src_hbm, dst_hbm, typ_hbm, table_hbm, sum_out,
             ebuf, srcbuf, segbuf, rows, gsem, acc):
        c = lax.axis_index("c")
        s = lax.axis_index("s")
        lo = c * _NHALF
        lane = lax.iota(jnp.int32, 16)

        def zrow(i, _):
            for l in range(_D // 16):
                rows[i, pl.ds(l * 16, 16)] = jnp.zeros((16,), jnp.float32)
            return 0
        lax.fori_loop(0, _K, zrow, 0)
        for k in range(_ROWS_PER_TILE // _K):
            pltpu.sync_copy(rows,
                            acc.at[pl.ds(s * _ROWS_PER_TILE + k * _K, _K)])
        rem = _ROWS_PER_TILE % _K
        if rem:
            pltpu.sync_copy(
                rows.at[pl.ds(0, rem)],
                acc.at[pl.ds(s * _ROWS_PER_TILE
                             + (_ROWS_PER_TILE // _K) * _K, rem)])
        plsc.subcore_barrier()

        def chunk_body(i, _):
            base = (s + i * _NSUB) * _K
            pltpu.sync_copy(src_hbm.at[pl.ds(base, _K)], srcbuf)
            pltpu.sync_copy(dst_hbm.at[pl.ds(base, _K)], ebuf.at[0])
            pltpu.sync_copy(typ_hbm.at[pl.ds(base, _K)], ebuf.at[1])
            _seg_groups(ebuf, segbuf, lo, lane)
            pltpu.async_copy(table_hbm.at[srcbuf], rows, gsem).wait()
            pltpu.sync_copy(rows, acc.at[segbuf.at[0]], add=True)
            return 0

        lax.fori_loop(0, n_chunks_per_tile, chunk_body, 0)
        plsc.subcore_barrier()

        pltpu.sync_copy(
            acc.at[pl.ds(s * _ROWS_PER_TILE, _ROWS_PER_TILE)],
            sum_out.at[c, pl.ds(s * _ROWS_PER_TILE, _ROWS_PER_TILE)])

    mesh = plsc.VectorSubcoreMesh(core_axis_name="c", subcore_axis_name="s")
    return pl.kernel(
        body,
        out_type=jax.ShapeDtypeStruct((_NCORE, _ACC_ROWS, _D), jnp.float32),
        mesh=mesh, scratch_types=scratch)


def _make_dense(relu, n, block):
    grid = (n // block,)

    def body(xb, s0, s1, s2, c0, c1, c2, w0, w1, w2, rt, bb, ob):
        acc = jnp.dot(xb[...], rt[...], preferred_element_type=jnp.float32)
        acc = acc + jnp.dot(s0[...] / jnp.maximum(c0[...], 1.0), w0[...],
                            preferred_element_type=jnp.float32)
        acc = acc + jnp.dot(s1[...] / jnp.maximum(c1[...], 1.0), w1[...],
                            preferred_element_type=jnp.float32)
        acc = acc + jnp.dot(s2[...] / jnp.maximum(c2[...], 1.0), w2[...],
                            preferred_element_type=jnp.float32)
        acc = acc + bb[...]
        if relu:
            acc = jnp.maximum(acc, 0.0)
        ob[...] = acc

    node_spec = pl.BlockSpec((block, _D), lambda i: (i, 0))
    cnt_spec = pl.BlockSpec((block, 1), lambda i: (i, 0))
    w_spec = pl.BlockSpec((_D, _D), lambda i: (0, 0))
    b_spec = pl.BlockSpec((1, _D), lambda i: (0, 0))
    return pl.pallas_call(
        body,
        grid=grid,
        in_specs=[node_spec, node_spec, node_spec, node_spec,
                  cnt_spec, cnt_spec, cnt_spec,
                  w_spec, w_spec, w_spec, w_spec, b_spec],
        out_specs=node_spec,
        out_shape=jax.ShapeDtypeStruct((n, _D), jnp.float32),
    )


def kernel(x, edge_index, edge_type, W1, root1, b1, W2, root2, b2):
    e = edge_index.shape[1]
    chunks_total = -(-e // _K)
    n_chunks_per_tile = -(-chunks_total // _NSUB)
    e_pad = n_chunks_per_tile * _NSUB * _K
    pad = e_pad - e

    src = edge_index[0]
    dst = edge_index[1]
    typ = edge_type
    if pad:
        src = jnp.concatenate([src, jnp.zeros((pad,), jnp.int32)])
        dst = jnp.concatenate([dst, jnp.full((pad,), -1, jnp.int32)])
        typ = jnp.concatenate([typ, jnp.zeros((pad,), jnp.int32)])

    sc_sums = _make_sc_sums(n_chunks_per_tile)

    def split_sums(sums):
        return [sums[:, r * _SREL:r * _SREL + _NHALF, :].reshape(_N, _D)
                for r in range(_R)]

    sc_counts = _make_sc_counts(n_chunks_per_tile)
    cnt_raw = sc_counts(dst, typ)
    cnts = [cnt_raw[:, r * _SREL:r * _SREL + _NHALF, 0].reshape(_N, 1)
            for r in range(_R)]
    src = src + (cnt_raw[0, 0, :1].astype(jnp.int32) * 0)

    dense_relu = _make_dense(True, _N, 1000)
    dense = _make_dense(False, _N, 1000)

    sum1 = sc_sums(src, dst, typ, x)
    s1 = split_sums(sum1)
    h1 = dense_relu(x, s1[0], s1[1], s1[2], cnts[0], cnts[1], cnts[2],
                    W1[0], W1[1], W1[2], root1, b1.reshape(1, _D))

    sum2 = sc_sums(src, dst, typ, h1)
    s2 = split_sums(sum2)
    out = dense(h1, s2[0], s2[1], s2[2], cnts[0], cnts[1], cnts[2],
                W2[0], W2[1], W2[2], root2, b2.reshape(1, _D))
    return out


# speedup vs baseline: 8.0285x; 1.4445x over previous
import jax
import jax.numpy as jnp
from jax import lax
from jax.experimental import pallas as pl
from jax.experimental.pallas import tpu as pltpu
from jax.experimental.pallas import tpu_sc as plsc

_N = 10000
_R = 3
_D = 128
_K = 32
_NSUB = 16
_NCORE = 2
_NHALF = _N // _NCORE
_SREL = _NHALF + 8
_ACC_RAW = _R * _SREL
_ACC_ROWS = ((_ACC_RAW + 64 + 127) // 128) * 128
_TRASH0 = _ACC_RAW
_ROWS_PER_TILE = _ACC_ROWS // _NSUB


def _seg_chunk(ebuf, segbuf, lo, lane):
    for g in range(_K // 16):
        d = ebuf[1, pl.ds(g * 16, 16)]
        t = ebuf[2, pl.ds(g * 16, 16)]
        dl = d - lo
        inrange = (dl >= 0) & (dl < _NHALF)
        trash = _TRASH0 + ((lane + g * 16) & 63)
        segbuf[0, pl.ds(g * 16, 16)] = jnp.where(inrange, t * _SREL + dl,
                                                 trash)


def _zero_rows(rows, n):
    def zrow(i, _):
        for l in range(_D // 16):
            rows[i, pl.ds(l * 16, 16)] = jnp.zeros((16,), jnp.float32)
        return 0
    lax.fori_loop(0, n, zrow, 0)


def _init_acc(rows, acc, s):
    for k in range(_ROWS_PER_TILE // _K):
        pltpu.sync_copy(rows, acc.at[pl.ds(s * _ROWS_PER_TILE + k * _K, _K)])
    rem = _ROWS_PER_TILE % _K
    if rem:
        pltpu.sync_copy(
            rows.at[pl.ds(0, rem)],
            acc.at[pl.ds(s * _ROWS_PER_TILE + (_ROWS_PER_TILE // _K) * _K,
                         rem)])


def _make_sc_sums(n_chunks_per_tile):
    n2 = n_chunks_per_tile // 2
    assert n_chunks_per_tile % 2 == 0 and n_chunks_per_tile >= 4

    scratch = [
        pltpu.VMEM((3, _K), jnp.int32),
        pltpu.VMEM((3, _K), jnp.int32),
        pltpu.VMEM((1, _K), jnp.int32),
        pltpu.VMEM((1, _K), jnp.int32),
        pltpu.VMEM((_K, _D), jnp.float32),
        pltpu.VMEM((_K, _D), jnp.float32),
        pltpu.SemaphoreType.DMA,
        pltpu.SemaphoreType.DMA,
        pltpu.VMEM_SHARED((_ACC_ROWS, _D), jnp.float32),
    ]

    def body(edges_hbm, table_hbm, sum_out,
             eb0, eb1, sg0, sg1, rw0, rw1, sem0, sem1, acc):
        c = lax.axis_index("c")
        s = lax.axis_index("s")
        lo = c * _NHALF
        lane = lax.iota(jnp.int32, 16)
        eb = (eb0, eb1)
        sg = (sg0, sg1)
        rw = (rw0, rw1)
        sem = (sem0, sem1)

        _zero_rows(rw0, _K)
        _init_acc(rw0, acc, s)
        plsc.subcore_barrier()

        def issue(slot, ch):
            pltpu.sync_copy(edges_hbm.at[ch], eb[slot])
            _seg_chunk(eb[slot], sg[slot], lo, lane)
            return pltpu.async_copy(table_hbm.at[eb[slot].at[0]], rw[slot],
                                    sem[slot])

        def drain(slot):
            pltpu.make_async_copy(table_hbm.at[eb[slot].at[0]], rw[slot],
                                  sem[slot]).wait()

        def scatter(slot):
            pltpu.sync_copy(rw[slot], acc.at[sg[slot].at[0]], add=True)

        issue(0, s)

        def pair_body(i2, _):
            a = 2 * i2
            b = a + 1
            issue(1, s + b * _NSUB)
            drain(0)
            scatter(0)
            nxt = jnp.minimum(b + 1, n_chunks_per_tile - 1)
            issue(0, s + nxt * _NSUB)
            drain(1)
            scatter(1)
            return 0

        lax.fori_loop(0, n2, pair_body, 0)
        drain(0)
        plsc.subcore_barrier()

        pltpu.sync_copy(
            acc.at[pl.ds(s * _ROWS_PER_TILE, _ROWS_PER_TILE)],
            sum_out.at[c, pl.ds(s * _ROWS_PER_TILE, _ROWS_PER_TILE)])

    mesh = plsc.VectorSubcoreMesh(core_axis_name="c", subcore_axis_name="s")
    return pl.kernel(
        body,
        out_type=jax.ShapeDtypeStruct((_NCORE, _ACC_ROWS, _D), jnp.float32),
        mesh=mesh, scratch_types=scratch)


def _make_sc_counts(n_chunks_per_tile):
    scratch = [
        pltpu.VMEM((3, _K), jnp.int32),
        pltpu.VMEM((3, _K), jnp.int32),
        pltpu.VMEM((1, _K), jnp.int32),
        pltpu.VMEM((1, _K), jnp.int32),
        pltpu.VMEM((_K, _D), jnp.float32),
        pltpu.SemaphoreType.DMA,
        pltpu.SemaphoreType.DMA,
        pltpu.VMEM_SHARED((_ACC_ROWS, _D), jnp.float32),
    ]

    def body(edges_hbm, cnt_out, eb0, eb1, sg0, sg1, rows, sem0, sem1,
             cntacc):
        c = lax.axis_index("c")
        s = lax.axis_index("s")
        lo = c * _NHALF
        lane = lax.iota(jnp.int32, 16)
        eb = (eb0, eb1)
        sg = (sg0, sg1)
        sem = (sem0, sem1)

        _zero_rows(rows, _K)
        _init_acc(rows, cntacc, s)

        def orow(i, _):
            for l in range(_D // 16):
                rows[i, pl.ds(l * 16, 16)] = jnp.ones((16,), jnp.float32)
            return 0
        lax.fori_loop(0, _K, orow, 0)
        plsc.subcore_barrier()

        def load(slot, ch):
            return pltpu.async_copy(edges_hbm.at[ch], eb[slot], sem[slot])

        def drain(slot, ch):
            pltpu.make_async_copy(edges_hbm.at[ch], eb[slot],
                                  sem[slot]).wait()

        def scatter(slot):
            _seg_chunk(eb[slot], sg[slot], lo, lane)
            pltpu.sync_copy(rows, cntacc.at[sg[slot].at[0]], add=True)

        load(0, s)

        def pair_body(i2, _):
            a = 2 * i2
            b = a + 1
            load(1, s + b * _NSUB)
            drain(0, s)
            scatter(0)
            nxt = jnp.minimum(b + 1, n_chunks_per_tile - 1)
            load(0, s + nxt * _NSUB)
            drain(1, s)
            scatter(1)
            return 0

        lax.fori_loop(0, n_chunks_per_tile // 2, pair_body, 0)
        drain(0, s)
        plsc.subcore_barrier()

        pltpu.sync_copy(
            cntacc.at[pl.ds(s * _ROWS_PER_TILE, _ROWS_PER_TILE)],
            cnt_out.at[c, pl.ds(s * _ROWS_PER_TILE, _ROWS_PER_TILE)])

    mesh = plsc.VectorSubcoreMesh(core_axis_name="c", subcore_axis_name="s")
    return pl.kernel(
        body,
        out_type=jax.ShapeDtypeStruct((_NCORE, _ACC_ROWS, _D), jnp.float32),
        mesh=mesh, scratch_types=scratch)


def _make_dense(relu, n, block):
    grid = (n // block,)

    def body(xb, s0, s1, s2, c0, c1, c2, w0, w1, w2, rt, bb, ob):
        acc = jnp.dot(xb[...], rt[...], preferred_element_type=jnp.float32)
        acc = acc + jnp.dot(s0[...] / jnp.maximum(c0[...], 1.0), w0[...],
                            preferred_element_type=jnp.float32)
        acc = acc + jnp.dot(s1[...] / jnp.maximum(c1[...], 1.0), w1[...],
                            preferred_element_type=jnp.float32)
        acc = acc + jnp.dot(s2[...] / jnp.maximum(c2[...], 1.0), w2[...],
                            preferred_element_type=jnp.float32)
        acc = acc + bb[...]
        if relu:
            acc = jnp.maximum(acc, 0.0)
        ob[...] = acc

    node_spec = pl.BlockSpec((block, _D), lambda i: (i, 0))
    cnt_spec = pl.BlockSpec((block, 1), lambda i: (i, 0))
    w_spec = pl.BlockSpec((_D, _D), lambda i: (0, 0))
    b_spec = pl.BlockSpec((1, _D), lambda i: (0, 0))
    return pl.pallas_call(
        body,
        grid=grid,
        in_specs=[node_spec, node_spec, node_spec, node_spec,
                  cnt_spec, cnt_spec, cnt_spec,
                  w_spec, w_spec, w_spec, w_spec, b_spec],
        out_specs=node_spec,
        out_shape=jax.ShapeDtypeStruct((n, _D), jnp.float32),
    )


def kernel(x, edge_index, edge_type, W1, root1, b1, W2, root2, b2):
    e = edge_index.shape[1]
    chunks_total = -(-e // _K)
    n_chunks_per_tile = -(-chunks_total // _NSUB)
    if n_chunks_per_tile % 2:
        n_chunks_per_tile += 1
    e_pad = n_chunks_per_tile * _NSUB * _K
    pad = e_pad - e

    src = edge_index[0]
    dst = edge_index[1]
    typ = edge_type
    if pad:
        src = jnp.concatenate([src, jnp.zeros((pad,), jnp.int32)])
        dst = jnp.concatenate([dst, jnp.full((pad,), -1, jnp.int32)])
        typ = jnp.concatenate([typ, jnp.zeros((pad,), jnp.int32)])

    edges3 = (jnp.stack([src, dst, typ], axis=0)
              .reshape(3, e_pad // _K, _K).transpose(1, 0, 2))

    sc_sums = _make_sc_sums(n_chunks_per_tile)
    sc_counts = _make_sc_counts(n_chunks_per_tile)
    dense_relu = _make_dense(True, _N, 1000)
    dense = _make_dense(False, _N, 1000)

    def split_sums(sums):
        return [sums[:, r * _SREL:r * _SREL + _NHALF, :].reshape(_N, _D)
                for r in range(_R)]

    cnt_raw = sc_counts(edges3)
    cnts = [cnt_raw[:, r * _SREL:r * _SREL + _NHALF, 0].reshape(_N, 1)
            for r in range(_R)]
    edges3 = edges3 + (cnt_raw[0, 0, :1].astype(jnp.int32) * 0)

    sum1 = sc_sums(edges3, x)
    s1 = split_sums(sum1)
    h1 = dense_relu(x, s1[0], s1[1], s1[2], cnts[0], cnts[1], cnts[2],
                    W1[0], W1[1], W1[2], root1, b1.reshape(1, _D))

    sum2 = sc_sums(edges3, h1)
    s2 = split_sums(sum2)
    out = dense(h1, s2[0], s2[1], s2[2], cnts[0], cnts[1], cnts[2],
                W2[0], W2[1], W2[2], root2, b2.reshape(1, _D))
    return out


# speedup vs baseline: 8.2327x; 1.0254x over previous
import jax
import jax.numpy as jnp
from jax import lax
from jax.experimental import pallas as pl
from jax.experimental.pallas import tpu as pltpu
from jax.experimental.pallas import tpu_sc as plsc

_N = 10000
_R = 3
_D = 128
_K = 32
_NSUB = 16
_NCORE = 2
_NHALF = _N // _NCORE
_SREL = _NHALF + 8
_ACC_RAW = _R * _SREL
_ACC_ROWS = ((_ACC_RAW + 64 + 127) // 128) * 128
_TRASH0 = _ACC_RAW
_ROWS_PER_TILE = _ACC_ROWS // _NSUB


def _seg_chunk(ebuf, segbuf, lo, lane):
    for g in range(_K // 16):
        d = ebuf[1, pl.ds(g * 16, 16)]
        t = ebuf[2, pl.ds(g * 16, 16)]
        dl = d - lo
        inrange = (dl >= 0) & (dl < _NHALF)
        trash = _TRASH0 + ((lane + g * 16) & 63)
        segbuf[0, pl.ds(g * 16, 16)] = jnp.where(inrange, t * _SREL + dl,
                                                 trash)


def _zero_rows(rows, n, width=_D):
    def zrow(i, _):
        for l in range(width // 16):
            rows[i, pl.ds(l * 16, 16)] = jnp.zeros((16,), jnp.float32)
        return 0
    lax.fori_loop(0, n, zrow, 0)


def _init_acc(rows, acc, s):
    for k in range(_ROWS_PER_TILE // _K):
        pltpu.sync_copy(rows, acc.at[pl.ds(s * _ROWS_PER_TILE + k * _K, _K)])
    rem = _ROWS_PER_TILE % _K
    if rem:
        pltpu.sync_copy(
            rows.at[pl.ds(0, rem)],
            acc.at[pl.ds(s * _ROWS_PER_TILE + (_ROWS_PER_TILE // _K) * _K,
                         rem)])


def _make_sc_sums(n_chunks_per_tile):
    n2 = n_chunks_per_tile // 2
    assert n_chunks_per_tile % 2 == 0 and n_chunks_per_tile >= 4

    scratch = [
        pltpu.VMEM((3, _K), jnp.int32),
        pltpu.VMEM((3, _K), jnp.int32),
        pltpu.VMEM((1, _K), jnp.int32),
        pltpu.VMEM((1, _K), jnp.int32),
        pltpu.VMEM((_K, _D), jnp.float32),
        pltpu.VMEM((_K, _D), jnp.float32),
        pltpu.SemaphoreType.DMA,
        pltpu.SemaphoreType.DMA,
        pltpu.VMEM_SHARED((_ACC_ROWS, _D), jnp.float32),
    ]

    def body(edges_hbm, table_hbm, sum_out,
             eb0, eb1, sg0, sg1, rw0, rw1, sem0, sem1, acc):
        c = lax.axis_index("c")
        s = lax.axis_index("s")
        lo = c * _NHALF
        lane = lax.iota(jnp.int32, 16)
        eb = (eb0, eb1)
        sg = (sg0, sg1)
        rw = (rw0, rw1)
        sem = (sem0, sem1)

        _zero_rows(rw0, _K)
        _init_acc(rw0, acc, s)
        plsc.subcore_barrier()

        def issue(slot, ch):
            pltpu.sync_copy(edges_hbm.at[ch], eb[slot])
            _seg_chunk(eb[slot], sg[slot], lo, lane)
            return pltpu.async_copy(table_hbm.at[eb[slot].at[0]], rw[slot],
                                    sem[slot])

        def drain(slot):
            pltpu.make_async_copy(table_hbm.at[eb[slot].at[0]], rw[slot],
                                  sem[slot]).wait()

        def scatter(slot):
            pltpu.sync_copy(rw[slot], acc.at[sg[slot].at[0]], add=True)

        issue(0, s)

        def pair_body(i2, _):
            a = 2 * i2
            b = a + 1
            issue(1, s + b * _NSUB)
            drain(0)
            scatter(0)
            nxt = jnp.minimum(b + 1, n_chunks_per_tile - 1)
            issue(0, s + nxt * _NSUB)
            drain(1)
            scatter(1)
            return 0

        lax.fori_loop(0, n2, pair_body, 0)
        drain(0)
        plsc.subcore_barrier()

        pltpu.sync_copy(
            acc.at[pl.ds(s * _ROWS_PER_TILE, _ROWS_PER_TILE)],
            sum_out.at[c, pl.ds(s * _ROWS_PER_TILE, _ROWS_PER_TILE)])

    mesh = plsc.VectorSubcoreMesh(core_axis_name="c", subcore_axis_name="s")
    return pl.kernel(
        body,
        out_type=jax.ShapeDtypeStruct((_NCORE, _ACC_ROWS, _D), jnp.float32),
        mesh=mesh, scratch_types=scratch)


_CW = 32


def _make_sc_counts(n_chunks_per_tile):
    scratch = [
        pltpu.VMEM((3, _K), jnp.int32),
        pltpu.VMEM((3, _K), jnp.int32),
        pltpu.VMEM((1, _K), jnp.int32),
        pltpu.VMEM((1, _K), jnp.int32),
        pltpu.VMEM((_K, _CW), jnp.float32),
        pltpu.SemaphoreType.DMA,
        pltpu.SemaphoreType.DMA,
        pltpu.VMEM_SHARED((_ACC_ROWS, _CW), jnp.float32),
    ]

    def body(edges_hbm, cnt_out, eb0, eb1, sg0, sg1, rows, sem0, sem1,
             cntacc):
        c = lax.axis_index("c")
        s = lax.axis_index("s")
        lo = c * _NHALF
        lane = lax.iota(jnp.int32, 16)
        eb = (eb0, eb1)
        sg = (sg0, sg1)
        sem = (sem0, sem1)

        _zero_rows(rows, _K, _CW)
        _init_acc(rows, cntacc, s)

        def orow(i, _):
            for l in range(_CW // 16):
                rows[i, pl.ds(l * 16, 16)] = jnp.ones((16,), jnp.float32)
            return 0
        lax.fori_loop(0, _K, orow, 0)
        plsc.subcore_barrier()

        def load(slot, ch):
            return pltpu.async_copy(edges_hbm.at[ch], eb[slot], sem[slot])

        def drain(slot, ch):
            pltpu.make_async_copy(edges_hbm.at[ch], eb[slot],
                                  sem[slot]).wait()

        def scatter(slot):
            _seg_chunk(eb[slot], sg[slot], lo, lane)
            pltpu.sync_copy(rows, cntacc.at[sg[slot].at[0]], add=True)

        load(0, s)

        def pair_body(i2, _):
            a = 2 * i2
            b = a + 1
            load(1, s + b * _NSUB)
            drain(0, s)
            scatter(0)
            nxt = jnp.minimum(b + 1, n_chunks_per_tile - 1)
            load(0, s + nxt * _NSUB)
            drain(1, s)
            scatter(1)
            return 0

        lax.fori_loop(0, n_chunks_per_tile // 2, pair_body, 0)
        drain(0, s)
        plsc.subcore_barrier()

        pltpu.sync_copy(
            cntacc.at[pl.ds(s * _ROWS_PER_TILE, _ROWS_PER_TILE)],
            cnt_out.at[c, pl.ds(s * _ROWS_PER_TILE, _ROWS_PER_TILE)])

    mesh = plsc.VectorSubcoreMesh(core_axis_name="c", subcore_axis_name="s")
    return pl.kernel(
        body,
        out_type=jax.ShapeDtypeStruct((_NCORE, _ACC_ROWS, _CW), jnp.float32),
        mesh=mesh, scratch_types=scratch)


def _make_dense(relu, n, block):
    grid = (n // block,)

    def body(xb, s0, s1, s2, c0, c1, c2, w0, w1, w2, rt, bb, ob):
        acc = jnp.dot(xb[...], rt[...], preferred_element_type=jnp.float32)
        acc = acc + jnp.dot(s0[...] / jnp.maximum(c0[...], 1.0), w0[...],
                            preferred_element_type=jnp.float32)
        acc = acc + jnp.dot(s1[...] / jnp.maximum(c1[...], 1.0), w1[...],
                            preferred_element_type=jnp.float32)
        acc = acc + jnp.dot(s2[...] / jnp.maximum(c2[...], 1.0), w2[...],
                            preferred_element_type=jnp.float32)
        acc = acc + bb[...]
        if relu:
            acc = jnp.maximum(acc, 0.0)
        ob[...] = acc

    node_spec = pl.BlockSpec((block, _D), lambda i: (i, 0))
    cnt_spec = pl.BlockSpec((block, 1), lambda i: (i, 0))
    w_spec = pl.BlockSpec((_D, _D), lambda i: (0, 0))
    b_spec = pl.BlockSpec((1, _D), lambda i: (0, 0))
    return pl.pallas_call(
        body,
        grid=grid,
        in_specs=[node_spec, node_spec, node_spec, node_spec,
                  cnt_spec, cnt_spec, cnt_spec,
                  w_spec, w_spec, w_spec, w_spec, b_spec],
        out_specs=node_spec,
        out_shape=jax.ShapeDtypeStruct((n, _D), jnp.float32),
    )


def kernel(x, edge_index, edge_type, W1, root1, b1, W2, root2, b2):
    e = edge_index.shape[1]
    chunks_total = -(-e // _K)
    n_chunks_per_tile = -(-chunks_total // _NSUB)
    if n_chunks_per_tile % 2:
        n_chunks_per_tile += 1
    e_pad = n_chunks_per_tile * _NSUB * _K
    pad = e_pad - e

    src = edge_index[0]
    dst = edge_index[1]
    typ = edge_type
    if pad:
        src = jnp.concatenate([src, jnp.zeros((pad,), jnp.int32)])
        dst = jnp.concatenate([dst, jnp.full((pad,), -1, jnp.int32)])
        typ = jnp.concatenate([typ, jnp.zeros((pad,), jnp.int32)])

    edges3 = (jnp.stack([src, dst, typ], axis=0)
              .reshape(3, e_pad // _K, _K).transpose(1, 0, 2))

    sc_sums = _make_sc_sums(n_chunks_per_tile)
    sc_counts = _make_sc_counts(n_chunks_per_tile)
    dense_relu = _make_dense(True, _N, 1000)
    dense = _make_dense(False, _N, 1000)

    def split_sums(sums):
        return [sums[:, r * _SREL:r * _SREL + _NHALF, :].reshape(_N, _D)
                for r in range(_R)]

    cnt_raw = sc_counts(edges3)
    cnts = [cnt_raw[:, r * _SREL:r * _SREL + _NHALF, 0].reshape(_N, 1)
            for r in range(_R)]
    edges3 = edges3 + (cnt_raw[0, 0, :1].astype(jnp.int32) * 0)

    sum1 = sc_sums(edges3, x)
    s1 = split_sums(sum1)
    h1 = dense_relu(x, s1[0], s1[1], s1[2], cnts[0], cnts[1], cnts[2],
                    W1[0], W1[1], W1[2], root1, b1.reshape(1, _D))

    sum2 = sc_sums(edges3, h1)
    s2 = split_sums(sum2)
    out = dense(h1, s2[0], s2[1], s2[2], cnts[0], cnts[1], cnts[2],
                W2[0], W2[1], W2[2], root2, b2.reshape(1, _D))
    return out
